# baseline (speedup 1.0000x reference)
import jax
import jax.numpy as jnp
from jax import lax
from jax.experimental import pallas as pl
from jax.experimental.pallas import tpu as pltpu

_N = 471
_TOPK = 50
_ROWS = 4096
_CHUNKS = (64, 64, 64, 64, 64, 64, 64, 23)
_QR = 64
_NS_ITERS = 9
_ROUNDS = 10


def _body(xt_ref, qt_ref, w_ref, linw_ref, linb_ref, scal_ref, out_ref,
          xv_ref, lw_ref, sems, lw_sem):
    alpha = scal_ref[0]
    lamda = scal_ref[1]
    rho = scal_ref[2]
    mu = scal_ref[3]

    pltpu.make_async_copy(linw_ref, lw_ref, lw_sem).start()
    off = 0
    for c, rows in enumerate(_CHUNKS):
        pltpu.make_async_copy(
            xt_ref.at[pl.ds(off, rows), :],
            xv_ref.at[pl.ds(off, rows), :],
            sems.at[c]).start()
        off += rows

    qt = qt_ref[...]
    g = lax.dot_general(qt, qt, (((1,), (1,)), ((), ())),
                        preferred_element_type=jnp.float32)
    r = jnp.max(jnp.sum(jnp.abs(g), axis=1))
    eye = (jax.lax.broadcasted_iota(jnp.int32, (_QR, _QR), 0)
           == jax.lax.broadcasted_iota(jnp.int32, (_QR, _QR), 1))
    xinv = eye.astype(jnp.float32) * (1.0 / r)

    parts = []
    offs = []
    off = 0
    for rows in _CHUNKS:
        offs.append(off)
        off += rows

    def _chunk_sum(c):
        pltpu.make_async_copy(
            xt_ref.at[pl.ds(offs[c], _CHUNKS[c]), :],
            xv_ref.at[pl.ds(offs[c], _CHUNKS[c]), :],
            sems.at[c]).wait()
        parts.append(jnp.sum(xv_ref[pl.ds(offs[c], _CHUNKS[c]), :], axis=1,
                             keepdims=True))

    for k in range(_NS_ITERS):
        gx = jnp.dot(g, xinv, preferred_element_type=jnp.float32)
        xinv = 2.0 * xinv - jnp.dot(xinv, gx,
                                    preferred_element_type=jnp.float32)
        if k < len(_CHUNKS):
            _chunk_sum(k)
    for c in range(_NS_ITERS, len(_CHUNKS)):
        _chunk_sum(c)
    t1 = jnp.dot(xinv, qt, preferred_element_type=jnp.float32)
    m = (alpha / _N) * lax.dot_general(
        qt, t1, (((0,), (0,)), ((), ())),
        preferred_element_type=jnp.float32)

    a_col = jnp.concatenate(parts, axis=0)
    a_mean = a_col.reshape(1, _N) * (1.0 / _ROWS)

    pltpu.make_async_copy(linw_ref, lw_ref, lw_sem).wait()
    w = w_ref[...]
    w2 = lax.dot_general(a_mean, lw_ref[...], (((1,), (1,)), ((), ())),
                         preferred_element_type=jnp.float32) + linb_ref[...]
    ones = jnp.ones_like(w)

    z = jnp.zeros_like(w)
    v = rho * w
    for _ in range(_ROUNDS):
        b = w + lax.dot_general(v, m, (((1,), (0,)), ((), ())),
                                preferred_element_type=jnp.float32)
        grad = (w2 + v + rho * (2.0 * z - b - w)
                + (2.0 * lamda) * (jnp.sum(z) - 1.0) * ones)
        z2 = jnp.maximum(z - mu * grad, 0.0)
        zc = z2.reshape(_N, 1)
        rank = jnp.sum((zc > z2).astype(jnp.float32), axis=0, keepdims=True)
        z = jnp.where(rank < float(_TOPK), z2, 0.0)
        v = v + rho * (z - b)
    out_ref[...] = z / (jnp.sum(z) + 1e-8)


def kernel(x, q_t, w, b1, alpha, lamda, rho, mu, lin_W, lin_b):
    del b1
    xt = x.T
    w2d = w.reshape(1, _N)
    linb2d = lin_b.reshape(1, _N)
    scal = jnp.concatenate([alpha, lamda, rho, mu])
    smem = pl.BlockSpec(memory_space=pltpu.SMEM)
    anym = pl.BlockSpec(memory_space=pl.ANY)
    out = pl.pallas_call(
        _body,
        in_specs=[
            anym,
            pl.BlockSpec((_QR, _N), lambda: (0, 0)),
            pl.BlockSpec((1, _N), lambda: (0, 0)),
            anym,
            pl.BlockSpec((1, _N), lambda: (0, 0)),
            smem,
        ],
        out_specs=pl.BlockSpec((1, _N), lambda: (0, 0)),
        out_shape=jax.ShapeDtypeStruct((1, _N), jnp.float32),
        scratch_shapes=[pltpu.VMEM((_N, _ROWS), jnp.float32),
                        pltpu.VMEM((_N, _N), jnp.float32),
                        pltpu.SemaphoreType.DMA((len(_CHUNKS),)),
                        pltpu.SemaphoreType.DMA],
    )(xt, q_t, w2d, lin_W, linb2d, scal)
    return out.reshape(_N)
